# Initial kernel scaffold; baseline (speedup 1.0000x reference)
import functools

import jax
import jax.numpy as jnp
from jax import lax
from jax.experimental import pallas as pl
from jax.experimental.pallas import tpu as pltpu
from jax.experimental.pallas import tpu_sc as plsc

N = 10000
E = 320000
D = 128
ACC_W = 144

NC = 2
NS = 16
NW = NC * NS
EPW = E // NW
B = 80
CH = EPW // B
RPT = N // NS
ZR = 125


def _sc_body(x_hbm, row_hbm, col_hbm, ew_hbm, out_hbm,
             row_v, col_v, ew_v, gbuf, pbuf, zbuf, acc_sh, sem):
    c = lax.axis_index("c")
    s = lax.axis_index("s")
    wid = c * NS + s

    def zero_zbuf(i, _):
        for k in range(ACC_W // 16):
            zbuf[i, pl.ds(16 * k, 16)] = jnp.zeros((16,), jnp.float32)
        return _
    lax.fori_loop(0, ZR, zero_zbuf, None)
    for r in range(RPT // ZR):
        pltpu.sync_copy(zbuf, acc_sh.at[pl.ds(s * RPT + r * ZR, ZR)])
    plsc.subcore_barrier()

    pltpu.sync_copy(row_hbm.at[wid], row_v)
    pltpu.sync_copy(col_hbm.at[wid], col_v)
    pltpu.sync_copy(ew_hbm.at[wid], ew_v)

    def chunk(j, carry):
        pltpu.async_copy(x_hbm.at[row_v.at[j]], gbuf, sem).wait()

        def edge(e, carry2):
            w = ew_v[j, e]
            for k in range(D // 16):
                pbuf[e, pl.ds(16 * k, 16)] = gbuf[e, pl.ds(16 * k, 16)] * w
            lane = lax.iota(jnp.int32, 16)
            pbuf[e, pl.ds(D, 16)] = jnp.where(lane == 0, w, 0.0)
            return carry2
        lax.fori_loop(0, B, edge, 0)

        pltpu.sync_copy(pbuf, acc_sh.at[col_v.at[j]], add=True)
        return carry
    lax.fori_loop(0, CH, chunk, 0)

    plsc.subcore_barrier()

    pltpu.sync_copy(acc_sh.at[pl.ds(s * RPT, RPT)],
                    out_hbm.at[c, pl.ds(s * RPT, RPT)])


@jax.jit
def _sc_aggregate(x, row3, col3, ew3):
    mesh = plsc.VectorSubcoreMesh(core_axis_name="c", subcore_axis_name="s")
    return pl.kernel(
        _sc_body,
        out_type=jax.ShapeDtypeStruct((NC, N, ACC_W), jnp.float32),
        mesh=mesh,
        scratch_types=[
            pltpu.VMEM((CH, B), jnp.int32),
            pltpu.VMEM((CH, B), jnp.int32),
            pltpu.VMEM((CH, B), jnp.float32),
            pltpu.VMEM((B, D), jnp.float32),
            pltpu.VMEM((B, ACC_W), jnp.float32),
            pltpu.VMEM((ZR, ACC_W), jnp.float32),
            pltpu.VMEM_SHARED((N, ACC_W), jnp.float32),
            pltpu.SemaphoreType.DMA,
        ],
    )(x, row3, col3, ew3)


def _tc_body(x_ref, p_ref, wsT_ref, wnT_ref, b_ref, o_ref):
    x = x_ref[...]
    agg = p_ref[0, :, :D] + p_ref[1, :, :D]
    wsum = p_ref[0, :, D] + p_ref[1, :, D]
    neigh = agg / jnp.maximum(wsum, 1e-8)[:, None]
    h = (jnp.dot(x, wsT_ref[...], preferred_element_type=jnp.float32)
         + jnp.dot(neigh, wnT_ref[...], preferred_element_type=jnp.float32)
         + b_ref[...])
    nrm = jnp.sqrt(jnp.sum(h * h, axis=1, keepdims=True))
    o_ref[...] = h / jnp.maximum(nrm, 1e-12)


@jax.jit
def _tc_combine(x, partials, wsT, wnT, bias2d):
    R = 1000
    grid = (N // R,)
    return pl.pallas_call(
        _tc_body,
        grid=grid,
        in_specs=[
            pl.BlockSpec((R, D), lambda i: (i, 0)),
            pl.BlockSpec((NC, R, ACC_W), lambda i: (0, i, 0)),
            pl.BlockSpec((D, D), lambda i: (0, 0)),
            pl.BlockSpec((D, D), lambda i: (0, 0)),
            pl.BlockSpec((1, D), lambda i: (0, 0)),
        ],
        out_specs=pl.BlockSpec((R, D), lambda i: (i, 0)),
        out_shape=jax.ShapeDtypeStruct((N, D), jnp.float32),
    )(x, partials, wsT, wnT, bias2d)


def kernel(x, edge_index, edge_weight, W_self, W_neigh, bias):
    row3 = edge_index[0].reshape(NW, CH, B)
    col3 = edge_index[1].reshape(NW, CH, B)
    ew3 = edge_weight.reshape(NW, CH, B)
    partials = _sc_aggregate(x, row3, col3, ew3)
    return _tc_combine(x, partials, W_self.T, W_neigh.T,
                       bias.reshape(1, D))


# speedup vs baseline: 7.0045x; 7.0045x over previous
import jax
import jax.numpy as jnp
from jax import lax
from jax.experimental import pallas as pl
from jax.experimental.pallas import tpu as pltpu
from jax.experimental.pallas import tpu_sc as plsc

N = 10000
E = 320000
D = 128

NC = 2
NS = 16
NW = NC * NS
EPW = E // NW
B = 80
SL = 5
CHS = 25
NP = 10240
RPT = NP // NS


def _sc_body(x_hbm, row_hbm, col_hbm, ew_hbm, agg_hbm, ws_hbm,
             row_v, col_v, ew_v, gbuf, pbuf, ws_v, acc_sh, sem):
    c = lax.axis_index("c")
    s = lax.axis_index("s")
    wid = c * NS + s

    def zero_pbuf(i, _):
        for k in range(D // 16):
            pbuf[i, pl.ds(16 * k, 16)] = jnp.zeros((16,), jnp.float32)
        return _
    lax.fori_loop(0, B, zero_pbuf, None)

    def zero_ws(i, _):
        ws_v[pl.ds(i * 16, 16)] = jnp.zeros((16,), jnp.float32)
        return _
    lax.fori_loop(0, NP // 16, zero_ws, None)

    for r in range(RPT // B):
        pltpu.sync_copy(pbuf, acc_sh.at[pl.ds(s * RPT + r * B, B)])
    plsc.subcore_barrier()

    def slab(sl, carry):
        pltpu.sync_copy(row_hbm.at[wid, sl], row_v)
        pltpu.sync_copy(col_hbm.at[wid, sl], col_v)
        pltpu.sync_copy(ew_hbm.at[wid, sl], ew_v)

        def chunk(j, carry2):
            pltpu.async_copy(x_hbm.at[row_v.at[j]], gbuf, sem).wait()

            for g in range(B // 16):
                wv = ew_v[j, pl.ds(g * 16, 16)]
                iv = col_v[j, pl.ds(g * 16, 16)]
                plsc.addupdate_scatter(ws_v, [iv], wv)
                for l in range(16):
                    e = g * 16 + l
                    w = wv[l]
                    for k in range(D // 16):
                        pbuf[e, pl.ds(16 * k, 16)] = (
                            gbuf[e, pl.ds(16 * k, 16)] * w)

            pltpu.sync_copy(pbuf, acc_sh.at[col_v.at[j]], add=True)
            return carry2
        lax.fori_loop(0, CHS, chunk, 0)
        return carry
    lax.fori_loop(0, SL, slab, 0)

    plsc.subcore_barrier()

    pltpu.sync_copy(acc_sh.at[pl.ds(s * RPT, RPT)],
                    agg_hbm.at[c, pl.ds(s * RPT, RPT)])
    pltpu.sync_copy(ws_v, ws_hbm.at[wid])


@jax.jit
def _sc_aggregate(x, row4, col4, ew4):
    mesh = plsc.VectorSubcoreMesh(core_axis_name="c", subcore_axis_name="s")
    return pl.kernel(
        _sc_body,
        out_type=(
            jax.ShapeDtypeStruct((NC, NP, D), jnp.float32),
            jax.ShapeDtypeStruct((NW, NP), jnp.float32),
        ),
        mesh=mesh,
        compiler_params=pltpu.CompilerParams(needs_layout_passes=False),
        scratch_types=[
            pltpu.VMEM((CHS, B), jnp.int32),
            pltpu.VMEM((CHS, B), jnp.int32),
            pltpu.VMEM((CHS, B), jnp.float32),
            pltpu.VMEM((B, D), jnp.float32),
            pltpu.VMEM((B, D), jnp.float32),
            pltpu.VMEM((NP,), jnp.float32),
            pltpu.VMEM_SHARED((NP, D), jnp.float32),
            pltpu.SemaphoreType.DMA,
        ],
    )(x, row4, col4, ew4)


def _tc_body(x_ref, p_ref, w_ref, wsT_ref, wnT_ref, b_ref, o_ref):
    x = x_ref[...]
    agg = p_ref[0] + p_ref[1]
    wsum = jnp.sum(w_ref[...], axis=0)
    neigh = agg / jnp.maximum(wsum, 1e-8)[:, None]
    h = (jnp.dot(x, wsT_ref[...], preferred_element_type=jnp.float32)
         + jnp.dot(neigh, wnT_ref[...], preferred_element_type=jnp.float32)
         + b_ref[...])
    nrm = jnp.sqrt(jnp.sum(h * h, axis=1, keepdims=True))
    o_ref[...] = h / jnp.maximum(nrm, 1e-12)


@jax.jit
def _tc_combine(x, partials, wsums, wsT, wnT, bias2d):
    R = 1024
    grid = (pl.cdiv(N, R),)
    return pl.pallas_call(
        _tc_body,
        grid=grid,
        in_specs=[
            pl.BlockSpec((R, D), lambda i: (i, 0)),
            pl.BlockSpec((NC, R, D), lambda i: (0, i, 0)),
            pl.BlockSpec((NW, R), lambda i: (0, i)),
            pl.BlockSpec((D, D), lambda i: (0, 0)),
            pl.BlockSpec((D, D), lambda i: (0, 0)),
            pl.BlockSpec((1, D), lambda i: (0, 0)),
        ],
        out_specs=pl.BlockSpec((R, D), lambda i: (i, 0)),
        out_shape=jax.ShapeDtypeStruct((N, D), jnp.float32),
    )(x, partials, wsums, wsT, wnT, bias2d)


def kernel(x, edge_index, edge_weight, W_self, W_neigh, bias):
    row4 = edge_index[0].reshape(NW, SL, CHS, B)
    col4 = edge_index[1].reshape(NW, SL, CHS, B)
    ew4 = edge_weight.reshape(NW, SL, CHS, B)
    partials, wsums = _sc_aggregate(x, row4, col4, ew4)
    return _tc_combine(x, partials, wsums, W_self.T, W_neigh.T,
                       bias.reshape(1, D))
